# Initial kernel scaffold; baseline (speedup 1.0000x reference)
import jax
import jax.numpy as jnp
from jax.experimental import pallas as pl


def kernel(x, edge_index, Wp, bp, conv_ws, conv_bs, gammas, betas, W1, b1, W2, b2, W3, b3):
    raise NotImplementedError("write your pallas kernel here")


# speedup vs baseline: 14.8973x; 14.8973x over previous
import functools

import jax
import jax.numpy as jnp
from jax import lax
from jax.experimental import pallas as pl
from jax.experimental.pallas import tpu as pltpu
from jax.experimental.pallas import tpu_sc as plsc

N = 50000
E = 800000
H = 64
HH = H // 2

NSC = 2
NT = 16
CH = 128

_EPT_A = -(-E // (NT * CH)) * CH
E_PAD_A = _EPT_A * NT
J_A = _EPT_A // CH
SB = 17
NSB = J_A // SB
N_ACC = N + 48

_EPT_D = -(-E // (NSC * NT * CH)) * CH
E_PAD_D = _EPT_D * NSC * NT
J_D = _EPT_D // CH
N_DEG = 51200
DEG_TPW = N_DEG // NT

WB_MAIN = 3128
WB_LAST = N - 15 * WB_MAIN

R = 1000
GRID = N // R

_mesh = plsc.VectorSubcoreMesh(core_axis_name="c", subcore_axis_name="s",
                               num_cores=NSC, num_subcores=NT)
_sc_params = pltpu.CompilerParams(use_tc_tiling_on_sc=False)


@functools.partial(
    pl.kernel,
    out_type=jax.ShapeDtypeStruct((NSC, N_DEG), jnp.float32),
    mesh=_mesh,
    scratch_types=[
        pltpu.VMEM((14, CH), jnp.int32),
        pltpu.VMEM((CH,), jnp.float32),
        pltpu.VMEM((DEG_TPW,), jnp.float32),
        pltpu.VMEM_SHARED((N_DEG,), jnp.float32),
        pltpu.SemaphoreType.DMA,
    ],
    compiler_params=_sc_params,
)
def _sc_deg(dst_hbm, out_hbm, idx_v, ones_v, zero_v, acc, sem):
    c = lax.axis_index("c")
    s = lax.axis_index("s")
    w = c * NT + s

    for j in range(CH // 16):
        ones_v[pl.ds(j * 16, 16)] = jnp.ones((16,), jnp.float32)

    def zloop(j, _):
        zero_v[pl.ds(j * 16, 16)] = jnp.zeros((16,), jnp.float32)
        return 0

    lax.fori_loop(0, DEG_TPW // 16, zloop, 0)

    pltpu.sync_copy(zero_v, acc.at[pl.ds(s * DEG_TPW, DEG_TPW)])
    plsc.subcore_barrier()

    def outer(jj, _):
        pltpu.async_copy(dst_hbm.at[w, pl.ds(jj * 14, 14)], idx_v,
                         sem).wait()

        def body(j, _):
            pltpu.sync_copy(ones_v, acc.at[idx_v.at[j]], add=True)
            return 0

        lax.fori_loop(0, 14, body, 0)
        return 0

    lax.fori_loop(0, J_D // 14, outer, 0)
    plsc.subcore_barrier()

    pltpu.sync_copy(acc.at[pl.ds(s * DEG_TPW, DEG_TPW)],
                    out_hbm.at[c, pl.ds(s * DEG_TPW, DEG_TPW)])


@functools.partial(
    pl.kernel,
    out_type=jax.ShapeDtypeStruct((NSC, N, HH), jnp.float32),
    mesh=_mesh,
    scratch_types=[
        pltpu.VMEM((SB, CH), jnp.int32),
        pltpu.VMEM((SB, CH), jnp.int32),
        pltpu.VMEM((CH, HH), jnp.float32),
        pltpu.VMEM_SHARED((N_ACC, HH), jnp.float32),
        pltpu.SemaphoreType.DMA,
        pltpu.SemaphoreType.DMA,
    ],
    compiler_params=_sc_params,
)
def _sc_agg(table_hbm, src_hbm, dst_hbm, out_hbm, src_v, dst_v, rows_v, acc,
            sem, sem2):
    c = lax.axis_index("c")
    s = lax.axis_index("s")

    @pl.when(s < NT - 1)
    def _():
        r0 = s * WB_MAIN
        pltpu.async_copy(table_hbm.at[pl.ds(c * N + r0, WB_MAIN)],
                         acc.at[pl.ds(r0, WB_MAIN)], sem2).wait()

    @pl.when(s == NT - 1)
    def _():
        r0 = (NT - 1) * WB_MAIN
        pltpu.async_copy(table_hbm.at[pl.ds(c * N + r0, WB_LAST)],
                         acc.at[pl.ds(r0, WB_LAST)], sem2).wait()

    plsc.subcore_barrier()

    def outer(jj, _):
        pltpu.async_copy(src_hbm.at[c, s, pl.ds(jj * SB, SB)], src_v,
                         sem).wait()
        pltpu.async_copy(dst_hbm.at[s, pl.ds(jj * SB, SB)], dst_v,
                         sem).wait()

        def body(j, _):
            pltpu.async_copy(table_hbm.at[src_v.at[j]], rows_v, sem).wait()
            pltpu.sync_copy(rows_v, acc.at[dst_v.at[j]], add=True)
            return 0

        lax.fori_loop(0, SB, body, 0)
        return 0

    lax.fori_loop(0, NSB, outer, 0)
    plsc.subcore_barrier()

    @pl.when(s < NT - 1)
    def _():
        r0 = s * WB_MAIN
        pltpu.async_copy(acc.at[pl.ds(r0, WB_MAIN)],
                         out_hbm.at[c, pl.ds(r0, WB_MAIN)], sem2).wait()

    @pl.when(s == NT - 1)
    def _():
        r0 = (NT - 1) * WB_MAIN
        pltpu.async_copy(acc.at[pl.ds(r0, WB_LAST)],
                         out_hbm.at[c, pl.ds(r0, WB_LAST)], sem2).wait()


def _proj_body(x_ref, pdt_ref, wp_ref, bp_ref, w1_ref, tab_ref, dinv_ref):
    deg = pdt_ref[:, 0:1] + pdt_ref[:, 1:2] + 1.0
    dinv = lax.rsqrt(deg)
    h0 = jnp.maximum(
        jnp.dot(x_ref[...], wp_ref[...],
                preferred_element_type=jnp.float32) + bp_ref[...], 0.0)
    hw2 = jnp.dot(h0, w1_ref[...], preferred_element_type=jnp.float32) * dinv
    tab_ref[0] = hw2[:, :HH]
    tab_ref[1] = hw2[:, HH:]
    dinv_ref[...] = dinv


def _tc_proj(x, pd_t, Wp, bp, W1):
    return pl.pallas_call(
        _proj_body,
        grid=(GRID,),
        in_specs=[
            pl.BlockSpec((R, x.shape[1]), lambda i: (i, 0)),
            pl.BlockSpec((R, NSC), lambda i: (i, 0)),
            pl.BlockSpec(Wp.shape, lambda i: (0, 0)),
            pl.BlockSpec((1, H), lambda i: (0, 0)),
            pl.BlockSpec(W1.shape, lambda i: (0, 0)),
        ],
        out_specs=[
            pl.BlockSpec((NSC, R, HH), lambda i: (0, i, 0)),
            pl.BlockSpec((R, 1), lambda i: (i, 0)),
        ],
        out_shape=[
            jax.ShapeDtypeStruct((NSC, N, HH), jnp.float32),
            jax.ShapeDtypeStruct((N, 1), jnp.float32),
        ],
    )(x, pd_t, Wp, bp, W1)


def _post_body(a_ref, dinv_ref, b_ref, y_ref, stats_ref, acc_ref):
    i = pl.program_id(0)
    y = jnp.concatenate([a_ref[0], a_ref[1]], axis=1) * dinv_ref[...] \
        + b_ref[...]
    y_ref[...] = y

    @pl.when(i == 0)
    def _():
        acc_ref[...] = jnp.zeros_like(acc_ref)

    acc_ref[0:1, :] += jnp.sum(y, axis=0, keepdims=True)
    acc_ref[1:2, :] += jnp.sum(y * y, axis=0, keepdims=True)

    @pl.when(i == GRID - 1)
    def _():
        stats_ref[...] = acc_ref[...]


def _tc_post(A, dinv, b):
    return pl.pallas_call(
        _post_body,
        grid=(GRID,),
        in_specs=[
            pl.BlockSpec((NSC, R, HH), lambda i: (0, i, 0)),
            pl.BlockSpec((R, 1), lambda i: (i, 0)),
            pl.BlockSpec((1, H), lambda i: (0, 0)),
        ],
        out_specs=[
            pl.BlockSpec((R, H), lambda i: (i, 0)),
            pl.BlockSpec((2, H), lambda i: (0, 0)),
        ],
        out_shape=[
            jax.ShapeDtypeStruct((N, H), jnp.float32),
            jax.ShapeDtypeStruct((2, H), jnp.float32),
        ],
        scratch_shapes=[pltpu.VMEM((2, H), jnp.float32)],
    )(A, dinv, b)


def _next_body(has_res, y_ref, stats_ref, g_ref, be_ref, dinv_ref, wn_ref,
               hr_ref, h_ref, tab_ref):
    stats = stats_ref[...]
    m = stats[0:1, :] * (1.0 / N)
    var = stats[1:2, :] * (1.0 / N) - m * m
    rstd = lax.rsqrt(var + 1e-5)
    h = jnp.maximum((y_ref[...] - m) * rstd * g_ref[...] + be_ref[...], 0.0)
    if has_res:
        h = h + hr_ref[...]
    h_ref[...] = h
    hw2 = jnp.dot(h, wn_ref[...], preferred_element_type=jnp.float32) \
        * dinv_ref[...]
    tab_ref[0] = hw2[:, :HH]
    tab_ref[1] = hw2[:, HH:]


def _tc_next(y, stats, g, be, dinv, Wn, hr):
    has_res = hr is not None
    args = [y, stats, g, be, dinv, Wn]
    in_specs = [
        pl.BlockSpec((R, H), lambda i: (i, 0)),
        pl.BlockSpec((2, H), lambda i: (0, 0)),
        pl.BlockSpec((1, H), lambda i: (0, 0)),
        pl.BlockSpec((1, H), lambda i: (0, 0)),
        pl.BlockSpec((R, 1), lambda i: (i, 0)),
        pl.BlockSpec((H, H), lambda i: (0, 0)),
    ]
    if has_res:
        args.append(hr)
        in_specs.append(pl.BlockSpec((R, H), lambda i: (i, 0)))
        body = functools.partial(_next_body, True)
    else:
        def body(*a):
            return _next_body(False, *a[:6], None, *a[6:])
    return pl.pallas_call(
        body,
        grid=(GRID,),
        in_specs=in_specs,
        out_specs=[
            pl.BlockSpec((R, H), lambda i: (i, 0)),
            pl.BlockSpec((NSC, R, HH), lambda i: (0, i, 0)),
        ],
        out_shape=[
            jax.ShapeDtypeStruct((N, H), jnp.float32),
            jax.ShapeDtypeStruct((NSC, N, HH), jnp.float32),
        ],
    )(*args)


def _head_body(y_ref, stats_ref, g_ref, be_ref, hr_ref, w1_ref, b1_ref,
               w2_ref, b2_ref, w3_ref, b3_ref, out_ref):
    stats = stats_ref[...]
    m = stats[0:1, :] * (1.0 / N)
    var = stats[1:2, :] * (1.0 / N) - m * m
    rstd = lax.rsqrt(var + 1e-5)
    h = jnp.maximum((y_ref[...] - m) * rstd * g_ref[...] + be_ref[...], 0.0) \
        + hr_ref[...]
    h1 = jnp.maximum(
        jnp.dot(h, w1_ref[...], preferred_element_type=jnp.float32)
        + b1_ref[...], 0.0)
    h2 = jnp.maximum(
        jnp.dot(h1, w2_ref[...], preferred_element_type=jnp.float32)
        + b2_ref[...], 0.0)
    out_ref[...] = jnp.dot(h2, w3_ref[...],
                           preferred_element_type=jnp.float32) + b3_ref[...]


def _tc_head(y, stats, g, be, hr, W1, b1, W2, b2, W3, b3):
    nc = W3.shape[1]

    def full(a):
        return pl.BlockSpec(a.shape, lambda i: tuple(0 for _ in a.shape))

    b1r, b2r, b3r = b1.reshape(1, -1), b2.reshape(1, -1), b3.reshape(1, -1)
    return pl.pallas_call(
        _head_body,
        grid=(GRID,),
        in_specs=[
            pl.BlockSpec((R, H), lambda i: (i, 0)),
            pl.BlockSpec((2, H), lambda i: (0, 0)),
            pl.BlockSpec((1, H), lambda i: (0, 0)),
            pl.BlockSpec((1, H), lambda i: (0, 0)),
            pl.BlockSpec((R, H), lambda i: (i, 0)),
            full(W1), full(b1r), full(W2), full(b2r), full(W3), full(b3r),
        ],
        out_specs=pl.BlockSpec((R, nc), lambda i: (i, 0)),
        out_shape=jax.ShapeDtypeStruct((N, nc), jnp.float32),
    )(y, stats, g, be, hr, W1, b1r, W2, b2r, W3, b3r)


def kernel(x, edge_index, Wp, bp, conv_ws, conv_bs, gammas, betas,
           W1, b1, W2, b2, W3, b3):
    src = edge_index[0]
    dst = edge_index[1]

    ar_a = jnp.arange(E_PAD_A - E, dtype=jnp.int32)
    src_pad = ar_a % 512
    dst_pad_a = N + (ar_a % 48)
    src_a = jnp.concatenate([src, src_pad])
    dst_a = jnp.concatenate([dst, dst_pad_a]).reshape(NT, J_A, CH)
    src2 = jnp.stack([src_a, src_a + N]).reshape(NSC, NT, J_A, CH)

    ar_d = jnp.arange(E_PAD_D - E, dtype=jnp.int32)
    dst_d = jnp.concatenate([dst, N + (ar_d % 1024)]) \
        .reshape(NSC * NT, J_D, CH)

    pd = _sc_deg(dst_d)
    pd_t = jnp.transpose(pd[:, :N])

    tab, dinv = _tc_proj(x, pd_t, Wp, bp.reshape(1, H), conv_ws[0])

    h_prev = None
    for i in range(3):
        A = _sc_agg(tab.reshape(NSC * N, HH), src2, dst_a)
        y, stats = _tc_post(A, dinv, conv_bs[i].reshape(1, H))
        if i < 2:
            h_new, tab = _tc_next(y, stats, gammas[i].reshape(1, H),
                                  betas[i].reshape(1, H), dinv,
                                  conv_ws[i + 1], h_prev)
            h_prev = h_new
        else:
            logits = _tc_head(y, stats, gammas[i].reshape(1, H),
                              betas[i].reshape(1, H), h_prev,
                              W1, b1, W2, b2, W3, b3)
    return logits


# speedup vs baseline: 21.3813x; 1.4352x over previous
import functools

import jax
import jax.numpy as jnp
from jax import lax
from jax.experimental import pallas as pl
from jax.experimental.pallas import tpu as pltpu
from jax.experimental.pallas import tpu_sc as plsc

N = 50000
E = 800000
H = 64
HH = H // 2

NSC = 2
NT = 16
CH = 128

J_A = 392
_EPT_A = J_A * CH
E_PAD_A = _EPT_A * NT
SB = 14
NSB = J_A // SB
N_ACC = N + 48

_EPT_D = -(-E // (NSC * NT * CH)) * CH
E_PAD_D = _EPT_D * NSC * NT
J_D = _EPT_D // CH
N_DEG = 51200
DEG_TPW = N_DEG // NT

WB_MAIN = 3128
WB_LAST = N - 15 * WB_MAIN

R = 1000
GRID = N // R

_mesh = plsc.VectorSubcoreMesh(core_axis_name="c", subcore_axis_name="s",
                               num_cores=NSC, num_subcores=NT)
_sc_params = pltpu.CompilerParams(use_tc_tiling_on_sc=False)


@functools.partial(
    pl.kernel,
    out_type=jax.ShapeDtypeStruct((NSC, N_DEG), jnp.float32),
    mesh=_mesh,
    scratch_types=[
        pltpu.VMEM((14, CH), jnp.int32),
        pltpu.VMEM((CH,), jnp.float32),
        pltpu.VMEM((DEG_TPW,), jnp.float32),
        pltpu.VMEM_SHARED((N_DEG,), jnp.float32),
        pltpu.SemaphoreType.DMA,
    ],
    compiler_params=_sc_params,
)
def _sc_deg(dst_hbm, out_hbm, idx_v, ones_v, zero_v, acc, sem):
    c = lax.axis_index("c")
    s = lax.axis_index("s")
    w = c * NT + s

    for j in range(CH // 16):
        ones_v[pl.ds(j * 16, 16)] = jnp.ones((16,), jnp.float32)

    def zloop(j, _):
        zero_v[pl.ds(j * 16, 16)] = jnp.zeros((16,), jnp.float32)
        return 0

    lax.fori_loop(0, DEG_TPW // 16, zloop, 0)

    pltpu.sync_copy(zero_v, acc.at[pl.ds(s * DEG_TPW, DEG_TPW)])
    plsc.subcore_barrier()

    def outer(jj, _):
        pltpu.async_copy(dst_hbm.at[w, pl.ds(jj * 14, 14)], idx_v,
                         sem).wait()

        def body(j, _):
            pltpu.sync_copy(ones_v, acc.at[idx_v.at[j]], add=True)
            return 0

        lax.fori_loop(0, 14, body, 0)
        return 0

    lax.fori_loop(0, J_D // 14, outer, 0)
    plsc.subcore_barrier()

    pltpu.sync_copy(acc.at[pl.ds(s * DEG_TPW, DEG_TPW)],
                    out_hbm.at[c, pl.ds(s * DEG_TPW, DEG_TPW)])


@functools.partial(
    pl.kernel,
    out_type=jax.ShapeDtypeStruct((NSC, N, HH), jnp.float32),
    mesh=_mesh,
    scratch_types=[
        pltpu.VMEM((2, SB, CH), jnp.int32),
        pltpu.VMEM((2, SB, CH), jnp.int32),
        pltpu.VMEM((CH, HH), jnp.float32),
        pltpu.VMEM((CH, HH), jnp.float32),
        pltpu.VMEM_SHARED((N_ACC, HH), jnp.float32),
        pltpu.SemaphoreType.DMA,
        pltpu.SemaphoreType.DMA,
        pltpu.SemaphoreType.DMA,
        pltpu.SemaphoreType.DMA,
        pltpu.SemaphoreType.DMA,
    ],
    compiler_params=_sc_params,
)
def _sc_agg(table_hbm, src_hbm, dst_hbm, out_hbm, src_v, dst_v, rows0, rows1,
            acc, sem_g0, sem_g1, sem_is, sem_id, sem2):
    c = lax.axis_index("c")
    s = lax.axis_index("s")

    @pl.when(s < NT - 1)
    def _():
        r0 = s * WB_MAIN
        pltpu.async_copy(table_hbm.at[pl.ds(c * N + r0, WB_MAIN)],
                         acc.at[pl.ds(r0, WB_MAIN)], sem2).wait()

    @pl.when(s == NT - 1)
    def _():
        r0 = (NT - 1) * WB_MAIN
        pltpu.async_copy(table_hbm.at[pl.ds(c * N + r0, WB_LAST)],
                         acc.at[pl.ds(r0, WB_LAST)], sem2).wait()

    plsc.subcore_barrier()

    pltpu.async_copy(src_hbm.at[c, s, pl.ds(0, SB)], src_v.at[0],
                     sem_is).wait()
    pltpu.async_copy(dst_hbm.at[s, pl.ds(0, SB)], dst_v.at[0],
                     sem_id).wait()

    rows = (rows0, rows1)
    sems = (sem_g0, sem_g1)

    def outer(jj, _):
        p = lax.rem(jj, 2)

        @pl.when(jj > 0)
        def _():
            pltpu.make_async_copy(src_hbm.at[c, s, pl.ds(jj * SB, SB)],
                                  src_v.at[p], sem_is).wait()
            pltpu.make_async_copy(dst_hbm.at[s, pl.ds(jj * SB, SB)],
                                  dst_v.at[p], sem_id).wait()

        @pl.when(jj < NSB - 1)
        def _():
            pltpu.async_copy(src_hbm.at[c, s, pl.ds((jj + 1) * SB, SB)],
                             src_v.at[1 - p], sem_is)
            pltpu.async_copy(dst_hbm.at[s, pl.ds((jj + 1) * SB, SB)],
                             dst_v.at[1 - p], sem_id)

        pltpu.async_copy(table_hbm.at[src_v.at[p, 0]], rows0, sem_g0)
        for j in range(SB):
            cb = j % 2
            if j + 1 < SB:
                pltpu.async_copy(table_hbm.at[src_v.at[p, j + 1]],
                                 rows[1 - cb], sems[1 - cb])
            pltpu.make_async_copy(table_hbm.at[src_v.at[p, j]], rows[cb],
                                  sems[cb]).wait()
            pltpu.sync_copy(rows[cb], acc.at[dst_v.at[p, j]], add=True)
        return 0

    lax.fori_loop(0, NSB, outer, 0)
    plsc.subcore_barrier()

    @pl.when(s < NT - 1)
    def _():
        r0 = s * WB_MAIN
        pltpu.async_copy(acc.at[pl.ds(r0, WB_MAIN)],
                         out_hbm.at[c, pl.ds(r0, WB_MAIN)], sem2).wait()

    @pl.when(s == NT - 1)
    def _():
        r0 = (NT - 1) * WB_MAIN
        pltpu.async_copy(acc.at[pl.ds(r0, WB_LAST)],
                         out_hbm.at[c, pl.ds(r0, WB_LAST)], sem2).wait()


def _proj_body(x_ref, pdt_ref, wp_ref, bp_ref, w1_ref, tab_ref, dinv_ref):
    deg = pdt_ref[:, 0:1] + pdt_ref[:, 1:2] + 1.0
    dinv = lax.rsqrt(deg)
    h0 = jnp.maximum(
        jnp.dot(x_ref[...], wp_ref[...],
                preferred_element_type=jnp.float32) + bp_ref[...], 0.0)
    hw2 = jnp.dot(h0, w1_ref[...], preferred_element_type=jnp.float32) * dinv
    tab_ref[0] = hw2[:, :HH]
    tab_ref[1] = hw2[:, HH:]
    dinv_ref[...] = dinv


def _tc_proj(x, pd_t, Wp, bp, W1):
    return pl.pallas_call(
        _proj_body,
        grid=(GRID,),
        in_specs=[
            pl.BlockSpec((R, x.shape[1]), lambda i: (i, 0)),
            pl.BlockSpec((R, NSC), lambda i: (i, 0)),
            pl.BlockSpec(Wp.shape, lambda i: (0, 0)),
            pl.BlockSpec((1, H), lambda i: (0, 0)),
            pl.BlockSpec(W1.shape, lambda i: (0, 0)),
        ],
        out_specs=[
            pl.BlockSpec((NSC, R, HH), lambda i: (0, i, 0)),
            pl.BlockSpec((R, 1), lambda i: (i, 0)),
        ],
        out_shape=[
            jax.ShapeDtypeStruct((NSC, N, HH), jnp.float32),
            jax.ShapeDtypeStruct((N, 1), jnp.float32),
        ],
    )(x, pd_t, Wp, bp, W1)


def _post_body(a_ref, dinv_ref, b_ref, y_ref, stats_ref, acc_ref):
    i = pl.program_id(0)
    y = jnp.concatenate([a_ref[0], a_ref[1]], axis=1) * dinv_ref[...] \
        + b_ref[...]
    y_ref[...] = y

    @pl.when(i == 0)
    def _():
        acc_ref[...] = jnp.zeros_like(acc_ref)

    acc_ref[0:1, :] += jnp.sum(y, axis=0, keepdims=True)
    acc_ref[1:2, :] += jnp.sum(y * y, axis=0, keepdims=True)

    @pl.when(i == GRID - 1)
    def _():
        stats_ref[...] = acc_ref[...]


def _tc_post(A, dinv, b):
    return pl.pallas_call(
        _post_body,
        grid=(GRID,),
        in_specs=[
            pl.BlockSpec((NSC, R, HH), lambda i: (0, i, 0)),
            pl.BlockSpec((R, 1), lambda i: (i, 0)),
            pl.BlockSpec((1, H), lambda i: (0, 0)),
        ],
        out_specs=[
            pl.BlockSpec((R, H), lambda i: (i, 0)),
            pl.BlockSpec((2, H), lambda i: (0, 0)),
        ],
        out_shape=[
            jax.ShapeDtypeStruct((N, H), jnp.float32),
            jax.ShapeDtypeStruct((2, H), jnp.float32),
        ],
        scratch_shapes=[pltpu.VMEM((2, H), jnp.float32)],
    )(A, dinv, b)


def _next_body(has_res, y_ref, stats_ref, g_ref, be_ref, dinv_ref, wn_ref,
               hr_ref, h_ref, tab_ref):
    stats = stats_ref[...]
    m = stats[0:1, :] * (1.0 / N)
    var = stats[1:2, :] * (1.0 / N) - m * m
    rstd = lax.rsqrt(var + 1e-5)
    h = jnp.maximum((y_ref[...] - m) * rstd * g_ref[...] + be_ref[...], 0.0)
    if has_res:
        h = h + hr_ref[...]
    h_ref[...] = h
    hw2 = jnp.dot(h, wn_ref[...], preferred_element_type=jnp.float32) \
        * dinv_ref[...]
    tab_ref[0] = hw2[:, :HH]
    tab_ref[1] = hw2[:, HH:]


def _tc_next(y, stats, g, be, dinv, Wn, hr):
    has_res = hr is not None
    args = [y, stats, g, be, dinv, Wn]
    in_specs = [
        pl.BlockSpec((R, H), lambda i: (i, 0)),
        pl.BlockSpec((2, H), lambda i: (0, 0)),
        pl.BlockSpec((1, H), lambda i: (0, 0)),
        pl.BlockSpec((1, H), lambda i: (0, 0)),
        pl.BlockSpec((R, 1), lambda i: (i, 0)),
        pl.BlockSpec((H, H), lambda i: (0, 0)),
    ]
    if has_res:
        args.append(hr)
        in_specs.append(pl.BlockSpec((R, H), lambda i: (i, 0)))
        body = functools.partial(_next_body, True)
    else:
        def body(*a):
            return _next_body(False, *a[:6], None, *a[6:])
    return pl.pallas_call(
        body,
        grid=(GRID,),
        in_specs=in_specs,
        out_specs=[
            pl.BlockSpec((R, H), lambda i: (i, 0)),
            pl.BlockSpec((NSC, R, HH), lambda i: (0, i, 0)),
        ],
        out_shape=[
            jax.ShapeDtypeStruct((N, H), jnp.float32),
            jax.ShapeDtypeStruct((NSC, N, HH), jnp.float32),
        ],
    )(*args)


def _head_body(y_ref, stats_ref, g_ref, be_ref, hr_ref, w1_ref, b1_ref,
               w2_ref, b2_ref, w3_ref, b3_ref, out_ref):
    stats = stats_ref[...]
    m = stats[0:1, :] * (1.0 / N)
    var = stats[1:2, :] * (1.0 / N) - m * m
    rstd = lax.rsqrt(var + 1e-5)
    h = jnp.maximum((y_ref[...] - m) * rstd * g_ref[...] + be_ref[...], 0.0) \
        + hr_ref[...]
    h1 = jnp.maximum(
        jnp.dot(h, w1_ref[...], preferred_element_type=jnp.float32)
        + b1_ref[...], 0.0)
    h2 = jnp.maximum(
        jnp.dot(h1, w2_ref[...], preferred_element_type=jnp.float32)
        + b2_ref[...], 0.0)
    out_ref[...] = jnp.dot(h2, w3_ref[...],
                           preferred_element_type=jnp.float32) + b3_ref[...]


def _tc_head(y, stats, g, be, hr, W1, b1, W2, b2, W3, b3):
    nc = W3.shape[1]

    def full(a):
        return pl.BlockSpec(a.shape, lambda i: tuple(0 for _ in a.shape))

    b1r, b2r, b3r = b1.reshape(1, -1), b2.reshape(1, -1), b3.reshape(1, -1)
    return pl.pallas_call(
        _head_body,
        grid=(GRID,),
        in_specs=[
            pl.BlockSpec((R, H), lambda i: (i, 0)),
            pl.BlockSpec((2, H), lambda i: (0, 0)),
            pl.BlockSpec((1, H), lambda i: (0, 0)),
            pl.BlockSpec((1, H), lambda i: (0, 0)),
            pl.BlockSpec((R, H), lambda i: (i, 0)),
            full(W1), full(b1r), full(W2), full(b2r), full(W3), full(b3r),
        ],
        out_specs=pl.BlockSpec((R, nc), lambda i: (i, 0)),
        out_shape=jax.ShapeDtypeStruct((N, nc), jnp.float32),
    )(y, stats, g, be, hr, W1, b1r, W2, b2r, W3, b3r)


def kernel(x, edge_index, Wp, bp, conv_ws, conv_bs, gammas, betas,
           W1, b1, W2, b2, W3, b3):
    src = edge_index[0]
    dst = edge_index[1]

    ar_a = jnp.arange(E_PAD_A - E, dtype=jnp.int32)
    src_pad = ar_a % 512
    dst_pad_a = N + (ar_a % 48)
    src_a = jnp.concatenate([src, src_pad])
    dst_a = jnp.concatenate([dst, dst_pad_a]).reshape(NT, J_A, CH)
    src2 = jnp.stack([src_a, src_a + N]).reshape(NSC, NT, J_A, CH)

    ar_d = jnp.arange(E_PAD_D - E, dtype=jnp.int32)
    dst_d = jnp.concatenate([dst, N + (ar_d % 1024)]) \
        .reshape(NSC * NT, J_D, CH)

    pd = _sc_deg(dst_d)
    pd_t = jnp.transpose(pd[:, :N])

    tab, dinv = _tc_proj(x, pd_t, Wp, bp.reshape(1, H), conv_ws[0])

    h_prev = None
    for i in range(3):
        A = _sc_agg(tab.reshape(NSC * N, HH), src2, dst_a)
        y, stats = _tc_post(A, dinv, conv_bs[i].reshape(1, H))
        if i < 2:
            h_new, tab = _tc_next(y, stats, gammas[i].reshape(1, H),
                                  betas[i].reshape(1, H), dinv,
                                  conv_ws[i + 1], h_prev)
            h_prev = h_new
        else:
            logits = _tc_head(y, stats, gammas[i].reshape(1, H),
                              betas[i].reshape(1, H), h_prev,
                              W1, b1, W2, b2, W3, b3)
    return logits
